# Initial kernel scaffold; baseline (speedup 1.0000x reference)
import jax
import jax.numpy as jnp
from jax.experimental import pallas as pl


def kernel(q_sub, q_rel, hidden, edges, nodes, old_nodes_new_idx, batchsize, curvature, edge_rule, query_rule_pref, rela_embed, Ws_w, Wr_w, Wqr_w, Wqr_b, walpha_w, walpha_b, Wh_w, rule_attn_w, rule_attn_b, rule_msg_w, rule_msg_b):
    raise NotImplementedError("write your pallas kernel here")


# speedup vs baseline: 4.7029x; 4.7029x over previous
import functools

import jax
import jax.numpy as jnp
from jax import lax
from jax.experimental import pallas as pl
from jax.experimental.pallas import tpu as pltpu
from jax.experimental.pallas import tpu_sc as plsc

MIN_NORM = 1e-15
MAXL = 50.0
EPS = 0.004
MINC = 1e-6

NC, NS = 2, 16
NW = NC * NS
CG = 128
MW = 144


def _node_tables_body(curv_ref, h_ref, w_ref, p_ref, hx_ref):
    h = h_ref[...]
    p_ref[...] = lax.dot_general(h, w_ref[...], (((1,), (1,)), ((), ())),
                                 preferred_element_type=jnp.float32)
    c = jnp.maximum(curv_ref[0, 0], MINC)
    sc = jnp.sqrt(c)
    un = jnp.maximum(jnp.sqrt(jnp.sum(h * h, axis=1, keepdims=True)), MIN_NORM)
    g = jnp.tanh(jnp.clip(sc * un, -15.0, 15.0)) * h / (sc * un)
    gn = jnp.maximum(jnp.sqrt(jnp.sum(g * g, axis=1, keepdims=True)), MIN_NORM)
    maxn = (1.0 - EPS) / sc
    hx_ref[...] = jnp.where(gn > maxn, g / gn * maxn, g)


def _rela_tables_body(curv_ref, h_ref, wr_ref, wqr_ref, q_ref, r_ref, hy_ref):
    h = h_ref[...]
    q_ref[...] = lax.dot_general(h, wr_ref[...], (((1,), (1,)), ((), ())),
                                 preferred_element_type=jnp.float32)
    r_ref[...] = lax.dot_general(h, wqr_ref[...], (((1,), (1,)), ((), ())),
                                 preferred_element_type=jnp.float32)
    c = jnp.maximum(curv_ref[0, 0], MINC)
    sc = jnp.sqrt(c)
    un = jnp.maximum(jnp.sqrt(jnp.sum(h * h, axis=1, keepdims=True)), MIN_NORM)
    g = jnp.tanh(jnp.clip(sc * un, -15.0, 15.0)) * h / (sc * un)
    gn = jnp.maximum(jnp.sqrt(jnp.sum(g * g, axis=1, keepdims=True)), MIN_NORM)
    maxn = (1.0 - EPS) / sc
    hy_ref[...] = jnp.where(gn > maxn, g / gn * maxn, g)


def _node_tables(curv11, hidden, Ws_w, br):
    n, d = hidden.shape
    return pl.pallas_call(
        _node_tables_body,
        grid=(n // br,),
        in_specs=[
            pl.BlockSpec((1, 1), lambda i: (0, 0)),
            pl.BlockSpec((br, d), lambda i: (i, 0)),
            pl.BlockSpec(Ws_w.shape, lambda i: (0, 0)),
        ],
        out_specs=[
            pl.BlockSpec((br, Ws_w.shape[0]), lambda i: (i, 0)),
            pl.BlockSpec((br, d), lambda i: (i, 0)),
        ],
        out_shape=[
            jax.ShapeDtypeStruct((n, Ws_w.shape[0]), jnp.float32),
            jax.ShapeDtypeStruct((n, d), jnp.float32),
        ],
    )(curv11, hidden, Ws_w)


def _rela_tables(curv11, rela, Wr_w, Wqr_w, br):
    n, d = rela.shape
    a = Wr_w.shape[0]
    return pl.pallas_call(
        _rela_tables_body,
        grid=(n // br,),
        in_specs=[
            pl.BlockSpec((1, 1), lambda i: (0, 0)),
            pl.BlockSpec((br, d), lambda i: (i, 0)),
            pl.BlockSpec(Wr_w.shape, lambda i: (0, 0)),
            pl.BlockSpec(Wqr_w.shape, lambda i: (0, 0)),
        ],
        out_specs=[
            pl.BlockSpec((br, a), lambda i: (i, 0)),
            pl.BlockSpec((br, a), lambda i: (i, 0)),
            pl.BlockSpec((br, d), lambda i: (i, 0)),
        ],
        out_shape=[
            jax.ShapeDtypeStruct((n, a), jnp.float32),
            jax.ShapeDtypeStruct((n, a), jnp.float32),
            jax.ShapeDtypeStruct((n, d), jnp.float32),
        ],
    )(curv11, rela, Wr_w, Wqr_w)


def _make_gather_kernel(E, N, Vp, D, DR):
    ew = E // NW
    n_full = ew // CG
    tail = ew - n_full * CG
    mesh = plsc.VectorSubcoreMesh(core_axis_name="c", subcore_axis_name="s",
                                  num_cores=NC, num_subcores=NS)

    @functools.partial(
        pl.kernel,
        out_type=(
            jax.ShapeDtypeStruct((E, D), jnp.float32),
            jax.ShapeDtypeStruct((E, D), jnp.float32),
            jax.ShapeDtypeStruct((E, D), jnp.float32),
            jax.ShapeDtypeStruct((E, D), jnp.float32),
            jax.ShapeDtypeStruct((E, D), jnp.float32),
            jax.ShapeDtypeStruct((E, DR), jnp.float32),
        ),
        mesh=mesh,
        scratch_types=[
            pltpu.VMEM((CG,), jnp.int32),
            pltpu.VMEM((CG,), jnp.int32),
            pltpu.VMEM((CG,), jnp.int32),
            pltpu.VMEM((CG,), jnp.int32),
            pltpu.VMEM((CG, D), jnp.float32),
            pltpu.VMEM((CG, D), jnp.float32),
            pltpu.VMEM((CG, D), jnp.float32),
            pltpu.VMEM((CG, D), jnp.float32),
            pltpu.VMEM((CG, D), jnp.float32),
            pltpu.VMEM((CG, DR), jnp.float32),
            pltpu.SemaphoreType.DMA,
        ],
        compiler_params=pltpu.CompilerParams(use_tc_tiling_on_sc=False),
    )
    def gather_k(sub_h, rel_h, ridx_h, qrel_h, p_h, q_h, r_h, hx_h, hy_h, qrp_h,
                 pg_h, qg_h, rg_h, xg_h, yg_h, qrpg_h,
                 subv, relv, ridxv, qqv, bp, bq, br_, bx, by, bqrp, sem):
        wid = lax.axis_index("s") * NC + lax.axis_index("c")
        base = wid * ew

        def do_chunk(off, cg):
            sl = pl.ds(0, cg)
            pltpu.sync_copy(sub_h.at[pl.ds(off, cg)], subv.at[sl])
            pltpu.sync_copy(rel_h.at[pl.ds(off, cg)], relv.at[sl])
            pltpu.sync_copy(ridx_h.at[pl.ds(off, cg)], ridxv.at[sl])
            pltpu.async_copy(qrel_h.at[ridxv.at[sl]], qqv.at[sl], sem).wait()
            cps = [
                pltpu.async_copy(p_h.at[subv.at[sl]], bp.at[sl], sem),
                pltpu.async_copy(q_h.at[relv.at[sl]], bq.at[sl], sem),
                pltpu.async_copy(r_h.at[qqv.at[sl]], br_.at[sl], sem),
                pltpu.async_copy(hx_h.at[subv.at[sl]], bx.at[sl], sem),
                pltpu.async_copy(hy_h.at[relv.at[sl]], by.at[sl], sem),
                pltpu.async_copy(qrp_h.at[ridxv.at[sl]], bqrp.at[sl], sem),
            ]
            for cp in cps:
                cp.wait()
            ods = pl.ds(off, cg)
            wps = [
                pltpu.async_copy(bp.at[sl], pg_h.at[ods], sem),
                pltpu.async_copy(bq.at[sl], qg_h.at[ods], sem),
                pltpu.async_copy(br_.at[sl], rg_h.at[ods], sem),
                pltpu.async_copy(bx.at[sl], xg_h.at[ods], sem),
                pltpu.async_copy(by.at[sl], yg_h.at[ods], sem),
                pltpu.async_copy(bqrp.at[sl], qrpg_h.at[ods], sem),
            ]
            for cp in wps:
                cp.wait()

        def body(i, _):
            do_chunk(base + i * CG, CG)
            return 0

        lax.fori_loop(0, n_full, body, 0)
        if tail:
            do_chunk(base + n_full * CG, tail)

    return gather_k


def _edge_body(scal_ref, attnw_ref, attnb_ref, wqrb_ref, walw_ref, msgw_ref,
               pg_ref, qg_ref, rg_ref, xg_ref, yg_ref, qrp_ref, er_ref, out_ref):
    c = jnp.maximum(scal_ref[0, 0], MINC)
    walpha_b = scal_ref[0, 1]
    msg_b = scal_ref[0, 2]
    sc = jnp.sqrt(c)

    rc = jnp.clip(er_ref[...] * qrp_ref[...], -1.0, 1.0)
    t1 = lax.dot_general(rc, attnw_ref[...], (((1,), (1,)), ((), ())),
                         preferred_element_type=jnp.float32)
    scale = 2.0 * jax.nn.sigmoid(t1 + attnb_ref[...])
    base = pg_ref[...] + qg_ref[...] + rg_ref[...] + wqrb_ref[...]
    feat = scale * base
    logit = jnp.clip(
        jnp.sum(jax.nn.relu(feat) * walw_ref[...], axis=1, keepdims=True) + walpha_b,
        -MAXL, MAXL)
    w = jnp.exp(logit)
    gate = 2.0 * jax.nn.sigmoid(
        jnp.sum(rc * msgw_ref[...], axis=1, keepdims=True) + msg_b)

    x = xg_ref[...]
    y = yg_ref[...]
    x2 = jnp.sum(x * x, axis=1, keepdims=True)
    y2 = jnp.sum(y * y, axis=1, keepdims=True)
    xy = jnp.sum(x * y, axis=1, keepdims=True)
    den = 1.0 + 2.0 * c * xy + c * c * x2 * y2
    denc = jnp.maximum(den, MIN_NORM)
    f1 = (1.0 + 2.0 * c * xy + c * y2) / denc
    f2 = (1.0 - c * x2) / denc
    nm2 = f1 * f1 * x2 + 2.0 * f1 * f2 * xy + f2 * f2 * y2
    nm = jnp.sqrt(jnp.maximum(nm2, 0.0))
    nmc = jnp.maximum(nm, MIN_NORM)
    maxn = (1.0 - EPS) / sc
    p = jnp.where(nmc > maxn, maxn / nmc, 1.0)
    yn = jnp.maximum(p * nm, MIN_NORM)
    z = jnp.minimum(sc * yn, 1.0 - EPS)
    t = 0.5 * (jnp.log1p(z) - jnp.log1p(-z)) / (sc * yn)
    wg = w * gate
    wA = wg * (t * p * f1)
    wB = wg * (t * p * f2)
    m = wA * x + wB * y
    be = m.shape[0]
    out_ref[...] = jnp.concatenate(
        [m, w, jnp.zeros((be, MW - m.shape[1] - 1), jnp.float32)], axis=1)


def _edge_stage(scal, attn_w, attn_b, wqr_b, wal_w, msg_w,
                pg, qg, rg, xg, yg, qrpg, er, be):
    E, D = pg.shape
    DR = er.shape[1]
    full = lambda a: pl.BlockSpec(a.shape, lambda i: tuple(0 for _ in a.shape))
    blk = lambda d_: pl.BlockSpec((be, d_), lambda i: (i, 0))
    return pl.pallas_call(
        _edge_body,
        grid=(E // be,),
        in_specs=[full(scal), full(attn_w), full(attn_b), full(wqr_b),
                  full(wal_w), full(msg_w),
                  blk(D), blk(D), blk(D), blk(D), blk(D), blk(DR), blk(DR)],
        out_specs=pl.BlockSpec((be, MW), lambda i: (i, 0)),
        out_shape=jax.ShapeDtypeStruct((E, MW), jnp.float32),
    )(scal, attn_w, attn_b, wqr_b, wal_w, msg_w, pg, qg, rg, xg, yg, qrpg, er)


def _make_scatter_kernel(E, N):
    HN = N // NC
    ew = E // NS
    CB = 400
    SUB = 80
    n_sub = CB // SUB
    n_chunk = ew // CB
    assert n_chunk * CB == ew
    rows_lo = (HN // NS) // 8 * 8
    rows_hi = HN - rows_lo * (NS - 1) + 8
    mesh = plsc.VectorSubcoreMesh(core_axis_name="c", subcore_axis_name="s",
                                  num_cores=NC, num_subcores=NS)

    @functools.partial(
        pl.kernel,
        out_type=jax.ShapeDtypeStruct((N, MW), jnp.float32),
        mesh=mesh,
        scratch_types=[
            pltpu.VMEM((CB, MW), jnp.float32),
            pltpu.VMEM((n_sub, SUB), jnp.int32),
            pltpu.VMEM_SHARED((HN + 8, MW), jnp.float32),
            pltpu.SemaphoreType.DMA,
        ],
        compiler_params=pltpu.CompilerParams(use_tc_tiling_on_sc=False),
    )
    def scatter_k(m_h, obj2_h, zu_h, up_h, mb, objv, ush, sem):
        cid = lax.axis_index("c")
        sid = lax.axis_index("s")
        lo = cid * HN
        base = sid * ew

        @pl.when(sid < NS - 1)
        def _():
            rsl = pl.ds(sid * rows_lo, rows_lo)
            pltpu.sync_copy(zu_h.at[rsl], ush.at[rsl])

        @pl.when(sid == NS - 1)
        def _():
            rsl = pl.ds((NS - 1) * rows_lo, rows_hi)
            pltpu.sync_copy(zu_h.at[rsl], ush.at[rsl])

        plsc.subcore_barrier()

        def body(i, _):
            off = base + i * CB
            pltpu.sync_copy(m_h.at[pl.ds(off, CB)], mb)
            pltpu.sync_copy(obj2_h.at[pl.ds(off // SUB, n_sub)], objv)
            for j in range(n_sub):
                for k in range(SUB // 16):
                    o = objv[j, pl.ds(k * 16, 16)] - lo
                    ok = (o >= 0) & (o < HN)
                    objv[j, pl.ds(k * 16, 16)] = jnp.where(ok, o, HN)
                pltpu.sync_copy(mb.at[pl.ds(j * SUB, SUB)],
                                ush.at[objv.at[j]], add=True)
            return 0

        lax.fori_loop(0, n_chunk, body, 0)
        plsc.subcore_barrier()

        @pl.when(sid < NS - 1)
        def _():
            rsl = pl.ds(sid * rows_lo, rows_lo)
            pltpu.sync_copy(ush.at[rsl], up_h.at[pl.ds(lo + sid * rows_lo, rows_lo)])

        @pl.when(sid == NS - 1)
        def _():
            nlast = HN - (NS - 1) * rows_lo
            pltpu.sync_copy(ush.at[pl.ds((NS - 1) * rows_lo, nlast)],
                            up_h.at[pl.ds(lo + (NS - 1) * rows_lo, nlast)])

    return scatter_k


def _final_body(u_ref, whp_ref, out_ref):
    u = u_ref[...]
    asum = jnp.maximum(u[:, 128:129], MIN_NORM)
    o = lax.dot_general(u, whp_ref[...], (((1,), (0,)), ((), ())),
                        preferred_element_type=jnp.float32)
    out_ref[...] = o / asum


def _final_stage(upart, whp, br):
    N, _ = upart.shape
    D = whp.shape[1]
    return pl.pallas_call(
        _final_body,
        grid=(N // br,),
        in_specs=[
            pl.BlockSpec((br, MW), lambda i: (i, 0)),
            pl.BlockSpec(whp.shape, lambda i: (0, 0)),
        ],
        out_specs=pl.BlockSpec((br, D), lambda i: (i, 0)),
        out_shape=jax.ShapeDtypeStruct((N, D), jnp.float32),
    )(upart, whp)


def kernel(q_sub, q_rel, hidden, edges, nodes, old_nodes_new_idx, batchsize,
           curvature, edge_rule, query_rule_pref, rela_embed, Ws_w, Wr_w,
           Wqr_w, Wqr_b, walpha_w, walpha_b, Wh_w, rule_attn_w, rule_attn_b,
           rule_msg_w, rule_msg_b):
    E = edges.shape[0]
    N, D = hidden.shape
    V = rela_embed.shape[0]
    DR = edge_rule.shape[1]
    A = Ws_w.shape[0]
    Vp = 10240 if V <= 10240 else ((V + 1023) // 1024) * 1024

    sub = edges[:, 4]
    rel = edges[:, 2]
    obj = edges[:, 5]
    ridx = edges[:, 0]
    curv11 = curvature.reshape(1, 1)
    rela_pad = jnp.zeros((Vp, D), jnp.float32).at[:V].set(rela_embed)

    P, HX = _node_tables(curv11, hidden, Ws_w, br=1000)
    Q, R, HY = _rela_tables(curv11, rela_pad, Wr_w, Wqr_w, br=1024)

    gather_k = _make_gather_kernel(E, N, Vp, D, DR)
    pg, qg, rg, xg, yg, qrpg = gather_k(sub, rel, ridx, q_rel,
                                        P, Q, R, HX, HY, query_rule_pref)

    scal = jnp.concatenate([curvature, walpha_b, rule_msg_b,
                            jnp.zeros((1,), jnp.float32)]).reshape(1, 4)
    m_rows = _edge_stage(scal, rule_attn_w, rule_attn_b.reshape(1, A),
                         Wqr_b.reshape(1, A), walpha_w, rule_msg_w,
                         pg, qg, rg, xg, yg, qrpg, edge_rule, be=3200)

    scatter_k = _make_scatter_kernel(E, N)
    obj2 = obj.reshape(E // 80, 80)
    zu = jnp.zeros((N // NC + 8, MW), jnp.float32)
    upart = scatter_k(m_rows, obj2, zu)

    whp = jnp.zeros((MW, D), jnp.float32).at[:D].set(Wh_w.T)
    return _final_stage(upart, whp, br=1000)


# speedup vs baseline: 5.3162x; 1.1304x over previous
import functools

import jax
import jax.numpy as jnp
from jax import lax
from jax.experimental import pallas as pl
from jax.experimental.pallas import tpu as pltpu
from jax.experimental.pallas import tpu_sc as plsc

MIN_NORM = 1e-15
MAXL = 50.0
EPS = 0.004
MINC = 1e-6

NC, NS = 2, 16
NW = NC * NS
CG = 128
MW = 128


def _node_tables_body(curv_ref, h_ref, w_ref, p_ref, hx_ref):
    h = h_ref[...]
    p_ref[...] = lax.dot_general(h, w_ref[...], (((1,), (1,)), ((), ())),
                                 preferred_element_type=jnp.float32)
    c = jnp.maximum(curv_ref[0, 0], MINC)
    sc = jnp.sqrt(c)
    un = jnp.maximum(jnp.sqrt(jnp.sum(h * h, axis=1, keepdims=True)), MIN_NORM)
    g = jnp.tanh(jnp.clip(sc * un, -15.0, 15.0)) * h / (sc * un)
    gn = jnp.maximum(jnp.sqrt(jnp.sum(g * g, axis=1, keepdims=True)), MIN_NORM)
    maxn = (1.0 - EPS) / sc
    hx_ref[...] = jnp.where(gn > maxn, g / gn * maxn, g)


def _rela_tables_body(curv_ref, h_ref, wr_ref, wqr_ref, q_ref, r_ref, hy_ref):
    h = h_ref[...]
    q_ref[...] = lax.dot_general(h, wr_ref[...], (((1,), (1,)), ((), ())),
                                 preferred_element_type=jnp.float32)
    r_ref[...] = lax.dot_general(h, wqr_ref[...], (((1,), (1,)), ((), ())),
                                 preferred_element_type=jnp.float32)
    c = jnp.maximum(curv_ref[0, 0], MINC)
    sc = jnp.sqrt(c)
    un = jnp.maximum(jnp.sqrt(jnp.sum(h * h, axis=1, keepdims=True)), MIN_NORM)
    g = jnp.tanh(jnp.clip(sc * un, -15.0, 15.0)) * h / (sc * un)
    gn = jnp.maximum(jnp.sqrt(jnp.sum(g * g, axis=1, keepdims=True)), MIN_NORM)
    maxn = (1.0 - EPS) / sc
    hy_ref[...] = jnp.where(gn > maxn, g / gn * maxn, g)


def _node_tables(curv11, hidden, Ws_w, br):
    n, d = hidden.shape
    return pl.pallas_call(
        _node_tables_body,
        grid=(n // br,),
        in_specs=[
            pl.BlockSpec((1, 1), lambda i: (0, 0)),
            pl.BlockSpec((br, d), lambda i: (i, 0)),
            pl.BlockSpec(Ws_w.shape, lambda i: (0, 0)),
        ],
        out_specs=[
            pl.BlockSpec((br, Ws_w.shape[0]), lambda i: (i, 0)),
            pl.BlockSpec((br, d), lambda i: (i, 0)),
        ],
        out_shape=[
            jax.ShapeDtypeStruct((n, Ws_w.shape[0]), jnp.float32),
            jax.ShapeDtypeStruct((n, d), jnp.float32),
        ],
    )(curv11, hidden, Ws_w)


def _rela_tables(curv11, rela, Wr_w, Wqr_w, br):
    n, d = rela.shape
    a = Wr_w.shape[0]
    return pl.pallas_call(
        _rela_tables_body,
        grid=(n // br,),
        in_specs=[
            pl.BlockSpec((1, 1), lambda i: (0, 0)),
            pl.BlockSpec((br, d), lambda i: (i, 0)),
            pl.BlockSpec(Wr_w.shape, lambda i: (0, 0)),
            pl.BlockSpec(Wqr_w.shape, lambda i: (0, 0)),
        ],
        out_specs=[
            pl.BlockSpec((br, a), lambda i: (i, 0)),
            pl.BlockSpec((br, a), lambda i: (i, 0)),
            pl.BlockSpec((br, d), lambda i: (i, 0)),
        ],
        out_shape=[
            jax.ShapeDtypeStruct((n, a), jnp.float32),
            jax.ShapeDtypeStruct((n, a), jnp.float32),
            jax.ShapeDtypeStruct((n, d), jnp.float32),
        ],
    )(curv11, rela, Wr_w, Wqr_w)


def _make_gather_kernel(E, N, Vp, D, DR):
    ew = E // NW
    n_full = ew // CG
    tail = ew - n_full * CG
    mesh = plsc.VectorSubcoreMesh(core_axis_name="c", subcore_axis_name="s",
                                  num_cores=NC, num_subcores=NS)

    @functools.partial(
        pl.kernel,
        out_type=(
            jax.ShapeDtypeStruct((E, D), jnp.float32),
            jax.ShapeDtypeStruct((E, D), jnp.float32),
            jax.ShapeDtypeStruct((E, D), jnp.float32),
            jax.ShapeDtypeStruct((E, D), jnp.float32),
        ),
        mesh=mesh,
        scratch_types=[
            pltpu.VMEM((CG,), jnp.int32),
            pltpu.VMEM((CG,), jnp.int32),
            pltpu.VMEM((CG,), jnp.int32),
            pltpu.VMEM((CG,), jnp.int32),
            pltpu.VMEM((CG, D), jnp.float32),
            pltpu.VMEM((CG, D), jnp.float32),
            pltpu.VMEM((CG, D), jnp.float32),
            pltpu.VMEM((CG, D), jnp.float32),
            pltpu.VMEM((CG, D), jnp.float32),
            pltpu.VMEM((CG, D), jnp.float32),
            pltpu.SemaphoreType.DMA,
        ],
        compiler_params=pltpu.CompilerParams(use_tc_tiling_on_sc=False),
    )
    def gather_k(sub_h, rel_h, ridx_h, qrel_h, p_h, q_h, r_h, hx_h, hy_h, qrp_h,
                 bg_h, xg_h, yg_h, qrpg_h,
                 subv, relv, ridxv, qqv, bp, bq, br_, bx, by, bqrp, sem):
        wid = lax.axis_index("s") * NC + lax.axis_index("c")
        base = wid * ew

        def do_chunk(off, cg):
            sl = pl.ds(0, cg)
            pltpu.sync_copy(sub_h.at[pl.ds(off, cg)], subv.at[sl])
            pltpu.sync_copy(rel_h.at[pl.ds(off, cg)], relv.at[sl])
            pltpu.sync_copy(ridx_h.at[pl.ds(off, cg)], ridxv.at[sl])
            pltpu.async_copy(qrel_h.at[ridxv.at[sl]], qqv.at[sl], sem).wait()
            cps = [
                pltpu.async_copy(p_h.at[subv.at[sl]], bp.at[sl], sem),
                pltpu.async_copy(q_h.at[relv.at[sl]], bq.at[sl], sem),
                pltpu.async_copy(r_h.at[qqv.at[sl]], br_.at[sl], sem),
                pltpu.async_copy(hx_h.at[subv.at[sl]], bx.at[sl], sem),
                pltpu.async_copy(hy_h.at[relv.at[sl]], by.at[sl], sem),
                pltpu.async_copy(qrp_h.at[ridxv.at[sl]], bqrp.at[sl], sem),
            ]
            for cp in cps:
                cp.wait()

            def addrow(r, _):
                for k in range(D // 16):
                    ls = pl.ds(k * 16, 16)
                    bp[r, ls] = bp[r, ls] + bq[r, ls] + br_[r, ls]
                return 0

            lax.fori_loop(0, cg, addrow, 0)

            ods = pl.ds(off, cg)
            wps = [
                pltpu.async_copy(bp.at[sl], bg_h.at[ods], sem),
                pltpu.async_copy(bx.at[sl], xg_h.at[ods], sem),
                pltpu.async_copy(by.at[sl], yg_h.at[ods], sem),
                pltpu.async_copy(bqrp.at[sl], qrpg_h.at[ods], sem),
            ]
            for cp in wps:
                cp.wait()

        def body(i, _):
            do_chunk(base + i * CG, CG)
            return 0

        lax.fori_loop(0, n_full, body, 0)
        if tail:
            do_chunk(base + n_full * CG, tail)

    return gather_k


def _edge_body(scal_ref, attnw_ref, attnb_ref, wqrb_ref, walw_ref, msgw_ref,
               bg_ref, xg_ref, yg_ref, qrp_ref, er_ref, out_ref, wout_ref):
    c = jnp.maximum(scal_ref[0, 0], MINC)
    walpha_b = scal_ref[0, 1]
    msg_b = scal_ref[0, 2]
    sc = jnp.sqrt(c)

    er = er_ref[...]
    be_, dr_ = er.shape
    rc = jnp.clip(er * qrp_ref[:, :dr_], -1.0, 1.0)
    t1 = lax.dot_general(rc, attnw_ref[...], (((1,), (1,)), ((), ())),
                         preferred_element_type=jnp.float32)
    scale = 2.0 * jax.nn.sigmoid(t1 + attnb_ref[...])
    base = bg_ref[...] + wqrb_ref[...]
    feat = scale * base
    logit = jnp.clip(
        jnp.sum(jax.nn.relu(feat) * walw_ref[...], axis=1, keepdims=True) + walpha_b,
        -MAXL, MAXL)
    w = jnp.exp(logit)
    gate = 2.0 * jax.nn.sigmoid(
        jnp.sum(rc * msgw_ref[...], axis=1, keepdims=True) + msg_b)

    x = xg_ref[...]
    y = yg_ref[...]
    x2 = jnp.sum(x * x, axis=1, keepdims=True)
    y2 = jnp.sum(y * y, axis=1, keepdims=True)
    xy = jnp.sum(x * y, axis=1, keepdims=True)
    cxy2 = 2.0 * c * xy
    a = 1.0 + cxy2 + c * y2
    b = 1.0 - c * x2
    denc = jnp.maximum(1.0 + cxy2 + (c * c) * (x2 * y2), MIN_NORM)
    rden = 1.0 / denc
    nm2 = (a * a) * x2 + (2.0 * a) * (b * xy) + (b * b) * y2
    nm = jnp.sqrt(jnp.maximum(nm2, 0.0)) * rden
    nmc = jnp.maximum(nm, MIN_NORM)
    maxn = (1.0 - EPS) / sc
    p = jnp.where(nmc > maxn, maxn / nmc, 1.0)
    yn = jnp.maximum(p * nm, MIN_NORM)
    z = jnp.minimum(sc * yn, 1.0 - EPS)
    t = 0.5 * (jnp.log1p(z) - jnp.log1p(-z)) / (sc * yn)
    wg = w * gate * t * p * rden
    wA = wg * a
    wB = wg * b
    out_ref[...] = wA * x + wB * y
    wout_ref[...] = jnp.reshape(w, (1, 1, be_))


def _edge_stage(scal, attn_w, attn_b, wqr_b, wal_w, msg_w,
                bg, xg, yg, qrpg, er, be):
    E, D = bg.shape
    DR = er.shape[1]
    full = lambda a: pl.BlockSpec(a.shape, lambda i: tuple(0 for _ in a.shape))
    blk = lambda d_: pl.BlockSpec((be, d_), lambda i: (i, 0))
    return pl.pallas_call(
        _edge_body,
        grid=(E // be,),
        in_specs=[full(scal), full(attn_w), full(attn_b), full(wqr_b),
                  full(wal_w), full(msg_w),
                  blk(D), blk(D), blk(D), blk(D), blk(DR)],
        out_specs=[pl.BlockSpec((be, MW), lambda i: (i, 0)),
                   pl.BlockSpec((1, 1, be), lambda i: (i, 0, 0))],
        out_shape=[jax.ShapeDtypeStruct((E, MW), jnp.float32),
                   jax.ShapeDtypeStruct((E // be, 1, be), jnp.float32)],
    )(scal, attn_w, attn_b, wqr_b, wal_w, msg_w, bg, xg, yg, qrpg, er)


def _make_scatter_kernel(E, N):
    HN = N // NC
    ew = E // NS
    CB = 400
    SUB = 80
    n_sub = CB // SUB
    n_chunk = ew // CB
    assert n_chunk * CB == ew
    rows_lo = (HN // NS) // 8 * 8
    rows_hi = HN - rows_lo * (NS - 1) + 8
    mesh = plsc.VectorSubcoreMesh(core_axis_name="c", subcore_axis_name="s",
                                  num_cores=NC, num_subcores=NS)

    @functools.partial(
        pl.kernel,
        out_type=(jax.ShapeDtypeStruct((N, MW), jnp.float32),
                  jax.ShapeDtypeStruct((N,), jnp.float32)),
        mesh=mesh,
        scratch_types=[
            pltpu.VMEM((CB, MW), jnp.float32),
            pltpu.VMEM((CB,), jnp.float32),
            pltpu.VMEM((n_sub, SUB), jnp.int32),
            pltpu.VMEM_SHARED((HN + 8, MW), jnp.float32),
            pltpu.VMEM_SHARED((HN + 8,), jnp.float32),
            pltpu.SemaphoreType.DMA,
        ],
        compiler_params=pltpu.CompilerParams(use_tc_tiling_on_sc=False),
    )
    def scatter_k(m_h, w_h, obj2_h, zu_h, za_h, up_h, ap_h, mb, wv, objv,
                  ush, ash, sem):
        cid = lax.axis_index("c")
        sid = lax.axis_index("s")
        lo = cid * HN
        base = sid * ew

        @pl.when(sid < NS - 1)
        def _():
            rsl = pl.ds(sid * rows_lo, rows_lo)
            pltpu.sync_copy(zu_h.at[rsl], ush.at[rsl])
            pltpu.sync_copy(za_h.at[rsl], ash.at[rsl])

        @pl.when(sid == NS - 1)
        def _():
            rsl = pl.ds((NS - 1) * rows_lo, rows_hi)
            pltpu.sync_copy(zu_h.at[rsl], ush.at[rsl])
            pltpu.sync_copy(za_h.at[rsl], ash.at[rsl])

        plsc.subcore_barrier()

        def body(i, _):
            off = base + i * CB
            pltpu.sync_copy(m_h.at[pl.ds(off, CB)], mb)
            pltpu.sync_copy(w_h.at[pl.ds(off, CB)], wv)
            pltpu.sync_copy(obj2_h.at[pl.ds(off // SUB, n_sub)], objv)
            for j in range(n_sub):
                for k in range(SUB // 16):
                    o = objv[j, pl.ds(k * 16, 16)] - lo
                    ok = (o >= 0) & (o < HN)
                    objv[j, pl.ds(k * 16, 16)] = jnp.where(ok, o, HN)
                pltpu.sync_copy(mb.at[pl.ds(j * SUB, SUB)],
                                ush.at[objv.at[j]], add=True)
                pltpu.sync_copy(wv.at[pl.ds(j * SUB, SUB)],
                                ash.at[objv.at[j]], add=True)
            return 0

        lax.fori_loop(0, n_chunk, body, 0)
        plsc.subcore_barrier()

        @pl.when(sid < NS - 1)
        def _():
            rsl = pl.ds(sid * rows_lo, rows_lo)
            osl = pl.ds(lo + sid * rows_lo, rows_lo)
            pltpu.sync_copy(ush.at[rsl], up_h.at[osl])
            pltpu.sync_copy(ash.at[rsl], ap_h.at[osl])

        @pl.when(sid == NS - 1)
        def _():
            nlast = HN - (NS - 1) * rows_lo
            rsl = pl.ds((NS - 1) * rows_lo, nlast)
            osl = pl.ds(lo + (NS - 1) * rows_lo, nlast)
            pltpu.sync_copy(ush.at[rsl], up_h.at[osl])
            pltpu.sync_copy(ash.at[rsl], ap_h.at[osl])

    return scatter_k


def _final_body(u_ref, a_ref, whp_ref, out_ref):
    u = u_ref[...]
    asum = jnp.maximum(jnp.reshape(a_ref[...], (u.shape[0], 1)), MIN_NORM)
    o = lax.dot_general(u, whp_ref[...], (((1,), (0,)), ((), ())),
                        preferred_element_type=jnp.float32)
    out_ref[...] = o / asum


def _final_stage(upart, apart, whp, br):
    N, _ = upart.shape
    D = whp.shape[1]
    a3 = apart.reshape(N // br, 1, br)
    return pl.pallas_call(
        _final_body,
        grid=(N // br,),
        in_specs=[
            pl.BlockSpec((br, MW), lambda i: (i, 0)),
            pl.BlockSpec((1, 1, br), lambda i: (i, 0, 0)),
            pl.BlockSpec(whp.shape, lambda i: (0, 0)),
        ],
        out_specs=pl.BlockSpec((br, D), lambda i: (i, 0)),
        out_shape=jax.ShapeDtypeStruct((N, D), jnp.float32),
    )(upart, a3, whp)


def kernel(q_sub, q_rel, hidden, edges, nodes, old_nodes_new_idx, batchsize,
           curvature, edge_rule, query_rule_pref, rela_embed, Ws_w, Wr_w,
           Wqr_w, Wqr_b, walpha_w, walpha_b, Wh_w, rule_attn_w, rule_attn_b,
           rule_msg_w, rule_msg_b):
    E = edges.shape[0]
    N, D = hidden.shape
    V = rela_embed.shape[0]
    DR = edge_rule.shape[1]
    A = Ws_w.shape[0]
    Vp = 10240 if V <= 10240 else ((V + 1023) // 1024) * 1024

    sub = edges[:, 4]
    rel = edges[:, 2]
    obj = edges[:, 5]
    ridx = edges[:, 0]
    curv11 = curvature.reshape(1, 1)
    rela_pad = jnp.zeros((Vp, D), jnp.float32).at[:V].set(rela_embed)

    P, HX = _node_tables(curv11, hidden, Ws_w, br=1000)
    Q, R, HY = _rela_tables(curv11, rela_pad, Wr_w, Wqr_w, br=1024)

    qrp_pad = jnp.zeros((N, D), jnp.float32).at[:, :DR].set(query_rule_pref)
    gather_k = _make_gather_kernel(E, N, Vp, D, DR)
    bg, xg, yg, qrpg = gather_k(sub, rel, ridx, q_rel,
                                P, Q, R, HX, HY, qrp_pad)

    scal = jnp.concatenate([curvature, walpha_b, rule_msg_b,
                            jnp.zeros((1,), jnp.float32)]).reshape(1, 4)
    m_rows, w3 = _edge_stage(scal, rule_attn_w, rule_attn_b.reshape(1, A),
                             Wqr_b.reshape(1, A), walpha_w, rule_msg_w,
                             bg, xg, yg, qrpg, edge_rule, be=3200)
    w_e = w3.reshape(E)

    scatter_k = _make_scatter_kernel(E, N)
    obj2 = obj.reshape(E // 80, 80)
    zu = jnp.zeros((N // NC + 8, MW), jnp.float32)
    za = jnp.zeros((N // NC + 8,), jnp.float32)
    upart, apart = scatter_k(m_rows, w_e, obj2, zu, za)

    return _final_stage(upart, apart, Wh_w.T, br=1000)


# speedup vs baseline: 6.6360x; 1.2483x over previous
import functools

import jax
import jax.numpy as jnp
from jax import lax
from jax.experimental import pallas as pl
from jax.experimental.pallas import tpu as pltpu
from jax.experimental.pallas import tpu_sc as plsc

MIN_NORM = 1e-15
MAXL = 50.0
EPS = 0.004
MINC = 1e-6

NC, NS = 2, 16
NW = NC * NS
CG = 128
MW = 128


def _node_tables_body(curv_ref, h_ref, w_ref, p_ref, hx_ref):
    h = h_ref[...]
    p_ref[...] = lax.dot_general(h, w_ref[...], (((1,), (1,)), ((), ())),
                                 preferred_element_type=jnp.float32)
    c = jnp.maximum(curv_ref[0, 0], MINC)
    sc = jnp.sqrt(c)
    un = jnp.maximum(jnp.sqrt(jnp.sum(h * h, axis=1, keepdims=True)), MIN_NORM)
    g = jnp.tanh(jnp.clip(sc * un, -15.0, 15.0)) * h / (sc * un)
    gn = jnp.maximum(jnp.sqrt(jnp.sum(g * g, axis=1, keepdims=True)), MIN_NORM)
    maxn = (1.0 - EPS) / sc
    hx_ref[...] = jnp.where(gn > maxn, g / gn * maxn, g)


def _rela_tables_body(curv_ref, h_ref, wr_ref, wqr_ref, q_ref, r_ref, hy_ref):
    h = h_ref[...]
    q_ref[...] = lax.dot_general(h, wr_ref[...], (((1,), (1,)), ((), ())),
                                 preferred_element_type=jnp.float32)
    r_ref[...] = lax.dot_general(h, wqr_ref[...], (((1,), (1,)), ((), ())),
                                 preferred_element_type=jnp.float32)
    c = jnp.maximum(curv_ref[0, 0], MINC)
    sc = jnp.sqrt(c)
    un = jnp.maximum(jnp.sqrt(jnp.sum(h * h, axis=1, keepdims=True)), MIN_NORM)
    g = jnp.tanh(jnp.clip(sc * un, -15.0, 15.0)) * h / (sc * un)
    gn = jnp.maximum(jnp.sqrt(jnp.sum(g * g, axis=1, keepdims=True)), MIN_NORM)
    maxn = (1.0 - EPS) / sc
    hy_ref[...] = jnp.where(gn > maxn, g / gn * maxn, g)


def _node_tables(curv11, hidden, Ws_w, br):
    n, d = hidden.shape
    return pl.pallas_call(
        _node_tables_body,
        grid=(n // br,),
        in_specs=[
            pl.BlockSpec((1, 1), lambda i: (0, 0)),
            pl.BlockSpec((br, d), lambda i: (i, 0)),
            pl.BlockSpec(Ws_w.shape, lambda i: (0, 0)),
        ],
        out_specs=[
            pl.BlockSpec((br, Ws_w.shape[0]), lambda i: (i, 0)),
            pl.BlockSpec((br, d), lambda i: (i, 0)),
        ],
        out_shape=[
            jax.ShapeDtypeStruct((n, Ws_w.shape[0]), jnp.float32),
            jax.ShapeDtypeStruct((n, d), jnp.float32),
        ],
    )(curv11, hidden, Ws_w)


def _rela_tables(curv11, rela, Wr_w, Wqr_w, br):
    n, d = rela.shape
    a = Wr_w.shape[0]
    return pl.pallas_call(
        _rela_tables_body,
        grid=(n // br,),
        in_specs=[
            pl.BlockSpec((1, 1), lambda i: (0, 0)),
            pl.BlockSpec((br, d), lambda i: (i, 0)),
            pl.BlockSpec(Wr_w.shape, lambda i: (0, 0)),
            pl.BlockSpec(Wqr_w.shape, lambda i: (0, 0)),
        ],
        out_specs=[
            pl.BlockSpec((br, a), lambda i: (i, 0)),
            pl.BlockSpec((br, a), lambda i: (i, 0)),
            pl.BlockSpec((br, d), lambda i: (i, 0)),
        ],
        out_shape=[
            jax.ShapeDtypeStruct((n, a), jnp.float32),
            jax.ShapeDtypeStruct((n, a), jnp.float32),
            jax.ShapeDtypeStruct((n, d), jnp.float32),
        ],
    )(curv11, rela, Wr_w, Wqr_w)


def _make_gather_kernel(E, N, Vp, D, DR):
    ew = E // NW
    n_full = ew // CG
    tail = ew - n_full * CG
    mesh = plsc.VectorSubcoreMesh(core_axis_name="c", subcore_axis_name="s",
                                  num_cores=NC, num_subcores=NS)

    @functools.partial(
        pl.kernel,
        out_type=(
            jax.ShapeDtypeStruct((E, D), jnp.float32),
            jax.ShapeDtypeStruct((E, D), jnp.float32),
            jax.ShapeDtypeStruct((E, D), jnp.float32),
            jax.ShapeDtypeStruct((E, D), jnp.float32),
        ),
        mesh=mesh,
        scratch_types=[
            pltpu.VMEM((CG,), jnp.int32),
            pltpu.VMEM((CG,), jnp.int32),
            pltpu.VMEM((CG,), jnp.int32),
            pltpu.VMEM((CG,), jnp.int32),
            pltpu.VMEM((CG, D), jnp.float32),
            pltpu.VMEM((CG, D), jnp.float32),
            pltpu.VMEM((CG, D), jnp.float32),
            pltpu.VMEM((CG, D), jnp.float32),
            pltpu.VMEM((CG, D), jnp.float32),
            pltpu.VMEM((CG, D), jnp.float32),
            pltpu.SemaphoreType.DMA,
        ],
        compiler_params=pltpu.CompilerParams(use_tc_tiling_on_sc=False),
    )
    def gather_k(sub_h, rel_h, ridx_h, qrel_h, p_h, q_h, r_h, hx_h, hy_h, qrp_h,
                 bg_h, xg_h, yg_h, qrpg_h,
                 subv, relv, ridxv, qqv, bp, bq, br_, bx, by, bqrp, sem):
        wid = lax.axis_index("s") * NC + lax.axis_index("c")
        base = wid * ew

        def do_chunk(off, cg):
            sl = pl.ds(0, cg)
            pltpu.sync_copy(sub_h.at[pl.ds(off, cg)], subv.at[sl])
            pltpu.sync_copy(rel_h.at[pl.ds(off, cg)], relv.at[sl])
            pltpu.sync_copy(ridx_h.at[pl.ds(off, cg)], ridxv.at[sl])
            pltpu.async_copy(qrel_h.at[ridxv.at[sl]], qqv.at[sl], sem).wait()
            cps = [
                pltpu.async_copy(p_h.at[subv.at[sl]], bp.at[sl], sem),
                pltpu.async_copy(q_h.at[relv.at[sl]], bq.at[sl], sem),
                pltpu.async_copy(r_h.at[qqv.at[sl]], br_.at[sl], sem),
                pltpu.async_copy(hx_h.at[subv.at[sl]], bx.at[sl], sem),
                pltpu.async_copy(hy_h.at[relv.at[sl]], by.at[sl], sem),
                pltpu.async_copy(qrp_h.at[ridxv.at[sl]], bqrp.at[sl], sem),
            ]
            for cp in cps:
                cp.wait()

            def addrow(r, _):
                for k in range(D // 16):
                    ls = pl.ds(k * 16, 16)
                    bp[r, ls] = bp[r, ls] + bq[r, ls] + br_[r, ls]
                return 0

            lax.fori_loop(0, cg, addrow, 0)

            ods = pl.ds(off, cg)
            wps = [
                pltpu.async_copy(bp.at[sl], bg_h.at[ods], sem),
                pltpu.async_copy(bx.at[sl], xg_h.at[ods], sem),
                pltpu.async_copy(by.at[sl], yg_h.at[ods], sem),
                pltpu.async_copy(bqrp.at[sl], qrpg_h.at[ods], sem),
            ]
            for cp in wps:
                cp.wait()

        def body(i, _):
            do_chunk(base + i * CG, CG)
            return 0

        lax.fori_loop(0, n_full, body, 0)
        if tail:
            do_chunk(base + n_full * CG, tail)

    return gather_k


def _edge_body(scal_ref, attnw_ref, attnb_ref, wqrb_ref, walw_ref, msgw_ref,
               bg_ref, xg_ref, yg_ref, qrp_ref, er_ref, out_ref, wout_ref):
    c = jnp.maximum(scal_ref[0, 0], MINC)
    walpha_b = scal_ref[0, 1]
    msg_b = scal_ref[0, 2]
    sc = jnp.sqrt(c)

    er = er_ref[...]
    be_, dr_ = er.shape
    rc = jnp.clip(er * qrp_ref[:, :dr_], -1.0, 1.0)
    t1 = lax.dot_general(rc, attnw_ref[...], (((1,), (1,)), ((), ())),
                         preferred_element_type=jnp.float32)
    scale = 2.0 * jax.nn.sigmoid(t1 + attnb_ref[...])
    base = bg_ref[...] + wqrb_ref[...]
    feat = scale * base
    logit = jnp.clip(
        jnp.sum(jax.nn.relu(feat) * walw_ref[...], axis=1, keepdims=True) + walpha_b,
        -MAXL, MAXL)
    w = jnp.exp(logit)
    gate = 2.0 * jax.nn.sigmoid(
        jnp.sum(rc * msgw_ref[...], axis=1, keepdims=True) + msg_b)

    x = xg_ref[...]
    y = yg_ref[...]
    x2 = jnp.sum(x * x, axis=1, keepdims=True)
    y2 = jnp.sum(y * y, axis=1, keepdims=True)
    xy = jnp.sum(x * y, axis=1, keepdims=True)
    cxy2 = 2.0 * c * xy
    a = 1.0 + cxy2 + c * y2
    b = 1.0 - c * x2
    denc = jnp.maximum(1.0 + cxy2 + (c * c) * (x2 * y2), MIN_NORM)
    rden = 1.0 / denc
    nm2 = (a * a) * x2 + (2.0 * a) * (b * xy) + (b * b) * y2
    nm = jnp.sqrt(jnp.maximum(nm2, 0.0)) * rden
    nmc = jnp.maximum(nm, MIN_NORM)
    maxn = (1.0 - EPS) / sc
    p = jnp.where(nmc > maxn, maxn / nmc, 1.0)
    yn = jnp.maximum(p * nm, MIN_NORM)
    z = jnp.minimum(sc * yn, 1.0 - EPS)
    t = 0.5 * (jnp.log1p(z) - jnp.log1p(-z)) / (sc * yn)
    wg = w * gate * t * p * rden
    wA = wg * a
    wB = wg * b
    out_ref[...] = wA * x + wB * y
    wout_ref[...] = jnp.reshape(w, (1, 1, be_))


def _edge_stage(scal, attn_w, attn_b, wqr_b, wal_w, msg_w,
                bg, xg, yg, qrpg, er, be):
    E, D = bg.shape
    DR = er.shape[1]
    full = lambda a: pl.BlockSpec(a.shape, lambda i: tuple(0 for _ in a.shape))
    blk = lambda d_: pl.BlockSpec((be, d_), lambda i: (i, 0))
    return pl.pallas_call(
        _edge_body,
        grid=(E // be,),
        in_specs=[full(scal), full(attn_w), full(attn_b), full(wqr_b),
                  full(wal_w), full(msg_w),
                  blk(D), blk(D), blk(D), blk(D), blk(DR)],
        out_specs=[pl.BlockSpec((be, MW), lambda i: (i, 0)),
                   pl.BlockSpec((1, 1, be), lambda i: (i, 0, 0))],
        out_shape=[jax.ShapeDtypeStruct((E, MW), jnp.float32),
                   jax.ShapeDtypeStruct((E // be, 1, be), jnp.float32)],
    )(scal, attn_w, attn_b, wqr_b, wal_w, msg_w, bg, xg, yg, qrpg, er)


def _make_scatter_kernel(E, N):
    HN = N // NC
    ew = E // NS
    CB = 400
    SUB = 80
    n_sub = CB // SUB
    n_chunk = ew // CB
    assert n_chunk * CB == ew
    rows_lo = (HN // NS) // 8 * 8
    rows_hi = HN - rows_lo * (NS - 1) + 8
    mesh = plsc.VectorSubcoreMesh(core_axis_name="c", subcore_axis_name="s",
                                  num_cores=NC, num_subcores=NS)

    @functools.partial(
        pl.kernel,
        out_type=(jax.ShapeDtypeStruct((N, MW), jnp.float32),
                  jax.ShapeDtypeStruct((N,), jnp.float32)),
        mesh=mesh,
        scratch_types=[
            pltpu.VMEM((CB, MW), jnp.float32),
            pltpu.VMEM((CB,), jnp.float32),
            pltpu.VMEM((n_sub, SUB), jnp.int32),
            pltpu.VMEM_SHARED((HN + 8, MW), jnp.float32),
            pltpu.VMEM_SHARED((HN + 8,), jnp.float32),
            pltpu.SemaphoreType.DMA,
        ],
        compiler_params=pltpu.CompilerParams(use_tc_tiling_on_sc=False),
    )
    def scatter_k(m_h, w_h, obj2_h, zu_h, za_h, up_h, ap_h, mb, wv, objv,
                  ush, ash, sem):
        cid = lax.axis_index("c")
        sid = lax.axis_index("s")
        lo = cid * HN
        base = sid * ew

        @pl.when(sid < NS - 1)
        def _():
            rsl = pl.ds(sid * rows_lo, rows_lo)
            pltpu.sync_copy(zu_h.at[rsl], ush.at[rsl])
            pltpu.sync_copy(za_h.at[rsl], ash.at[rsl])

        @pl.when(sid == NS - 1)
        def _():
            rsl = pl.ds((NS - 1) * rows_lo, rows_hi)
            pltpu.sync_copy(zu_h.at[rsl], ush.at[rsl])
            pltpu.sync_copy(za_h.at[rsl], ash.at[rsl])

        plsc.subcore_barrier()

        def body(i, _):
            off = base + i * CB
            pltpu.sync_copy(m_h.at[pl.ds(off, CB)], mb)
            pltpu.sync_copy(w_h.at[pl.ds(off, CB)], wv)
            pltpu.sync_copy(obj2_h.at[pl.ds(off // SUB, n_sub)], objv)
            for j in range(n_sub):
                for k in range(SUB // 16):
                    o = objv[j, pl.ds(k * 16, 16)] - lo
                    ok = (o >= 0) & (o < HN)
                    objv[j, pl.ds(k * 16, 16)] = jnp.where(ok, o, HN)
                pltpu.sync_copy(mb.at[pl.ds(j * SUB, SUB)],
                                ush.at[objv.at[j]], add=True)
                pltpu.sync_copy(wv.at[pl.ds(j * SUB, SUB)],
                                ash.at[objv.at[j]], add=True)
            return 0

        lax.fori_loop(0, n_chunk, body, 0)
        plsc.subcore_barrier()

        @pl.when(sid < NS - 1)
        def _():
            rsl = pl.ds(sid * rows_lo, rows_lo)
            osl = pl.ds(lo + sid * rows_lo, rows_lo)
            pltpu.sync_copy(ush.at[rsl], up_h.at[osl])
            pltpu.sync_copy(ash.at[rsl], ap_h.at[osl])

        @pl.when(sid == NS - 1)
        def _():
            nlast = HN - (NS - 1) * rows_lo
            rsl = pl.ds((NS - 1) * rows_lo, nlast)
            osl = pl.ds(lo + (NS - 1) * rows_lo, nlast)
            pltpu.sync_copy(ush.at[rsl], up_h.at[osl])
            pltpu.sync_copy(ash.at[rsl], ap_h.at[osl])

    return scatter_k


def _final_body(u0_ref, u1_ref, a0_ref, a1_ref, whp_ref, out_ref):
    u = u0_ref[...] + u1_ref[...]
    a = a0_ref[...] + a1_ref[...]
    asum = jnp.maximum(jnp.reshape(a, (u.shape[0], 1)), MIN_NORM)
    o = lax.dot_general(u, whp_ref[...], (((1,), (0,)), ((), ())),
                        preferred_element_type=jnp.float32)
    out_ref[...] = o / asum


def _final_stage(u0, u1, a0, a1, whp, br):
    N, _ = u0.shape
    D = whp.shape[1]
    a03 = a0.reshape(N // br, 1, br)
    a13 = a1.reshape(N // br, 1, br)
    return pl.pallas_call(
        _final_body,
        grid=(N // br,),
        in_specs=[
            pl.BlockSpec((br, MW), lambda i: (i, 0)),
            pl.BlockSpec((br, MW), lambda i: (i, 0)),
            pl.BlockSpec((1, 1, br), lambda i: (i, 0, 0)),
            pl.BlockSpec((1, 1, br), lambda i: (i, 0, 0)),
            pl.BlockSpec(whp.shape, lambda i: (0, 0)),
        ],
        out_specs=pl.BlockSpec((br, D), lambda i: (i, 0)),
        out_shape=jax.ShapeDtypeStruct((N, D), jnp.float32),
    )(u0, u1, a03, a13, whp)


def kernel(q_sub, q_rel, hidden, edges, nodes, old_nodes_new_idx, batchsize,
           curvature, edge_rule, query_rule_pref, rela_embed, Ws_w, Wr_w,
           Wqr_w, Wqr_b, walpha_w, walpha_b, Wh_w, rule_attn_w, rule_attn_b,
           rule_msg_w, rule_msg_b):
    E = edges.shape[0]
    N, D = hidden.shape
    V = rela_embed.shape[0]
    DR = edge_rule.shape[1]
    A = Ws_w.shape[0]
    Vp = 10240 if V <= 10240 else ((V + 1023) // 1024) * 1024

    sub = edges[:, 4]
    rel = edges[:, 2]
    obj = edges[:, 5]
    ridx = edges[:, 0]
    curv11 = curvature.reshape(1, 1)
    rela_pad = jnp.zeros((Vp, D), jnp.float32).at[:V].set(rela_embed)

    P, HX = _node_tables(curv11, hidden, Ws_w, br=1000)
    Q, R, HY = _rela_tables(curv11, rela_pad, Wr_w, Wqr_w, br=1024)

    qrp_pad = jnp.zeros((N, D), jnp.float32).at[:, :DR].set(query_rule_pref)
    scal = jnp.concatenate([curvature, walpha_b, rule_msg_b,
                            jnp.zeros((1,), jnp.float32)]).reshape(1, 4)
    E2 = E // 2
    gather_k = _make_gather_kernel(E2, N, Vp, D, DR)
    scatter_k = _make_scatter_kernel(E2, N)
    obj2 = obj.reshape(E // 80, 80)
    zu = jnp.zeros((N // NC + 8, MW), jnp.float32)
    za = jnp.zeros((N // NC + 8,), jnp.float32)

    parts = []
    for h in range(2):
        sl = slice(h * E2, (h + 1) * E2)
        bg, xg, yg, qrpg = gather_k(sub[sl], rel[sl], ridx[sl], q_rel,
                                    P, Q, R, HX, HY, qrp_pad)
        m_rows, w3 = _edge_stage(scal, rule_attn_w, rule_attn_b.reshape(1, A),
                                 Wqr_b.reshape(1, A), walpha_w, rule_msg_w,
                                 bg, xg, yg, qrpg, edge_rule[sl], be=3200)
        w_e = w3.reshape(E2)
        o2 = obj2[h * (E2 // 80):(h + 1) * (E2 // 80)]
        parts.append(scatter_k(m_rows, w_e, o2, zu, za))

    (u0, a0), (u1, a1) = parts
    return _final_stage(u0, u1, a0, a1, Wh_w.T, br=1000)


# speedup vs baseline: 6.7260x; 1.0136x over previous
import functools

import jax
import jax.numpy as jnp
from jax import lax
from jax.experimental import pallas as pl
from jax.experimental.pallas import tpu as pltpu
from jax.experimental.pallas import tpu_sc as plsc

MIN_NORM = 1e-15
MAXL = 50.0
EPS = 0.004
MINC = 1e-6

NC, NS = 2, 16
NW = NC * NS
CG = 128
MW = 128


def _node_tables_body(curv_ref, h_ref, w_ref, p_ref, hx_ref):
    h = h_ref[...]
    p_ref[...] = lax.dot_general(h, w_ref[...], (((1,), (1,)), ((), ())),
                                 preferred_element_type=jnp.float32)
    c = jnp.maximum(curv_ref[0, 0], MINC)
    sc = jnp.sqrt(c)
    un = jnp.maximum(jnp.sqrt(jnp.sum(h * h, axis=1, keepdims=True)), MIN_NORM)
    g = jnp.tanh(jnp.clip(sc * un, -15.0, 15.0)) * h / (sc * un)
    gn = jnp.maximum(jnp.sqrt(jnp.sum(g * g, axis=1, keepdims=True)), MIN_NORM)
    maxn = (1.0 - EPS) / sc
    hx_ref[...] = jnp.where(gn > maxn, g / gn * maxn, g)


def _rela_tables_body(curv_ref, h_ref, wr_ref, wqr_ref, q_ref, r_ref, hy_ref):
    h = h_ref[...]
    q_ref[...] = lax.dot_general(h, wr_ref[...], (((1,), (1,)), ((), ())),
                                 preferred_element_type=jnp.float32)
    r_ref[...] = lax.dot_general(h, wqr_ref[...], (((1,), (1,)), ((), ())),
                                 preferred_element_type=jnp.float32)
    c = jnp.maximum(curv_ref[0, 0], MINC)
    sc = jnp.sqrt(c)
    un = jnp.maximum(jnp.sqrt(jnp.sum(h * h, axis=1, keepdims=True)), MIN_NORM)
    g = jnp.tanh(jnp.clip(sc * un, -15.0, 15.0)) * h / (sc * un)
    gn = jnp.maximum(jnp.sqrt(jnp.sum(g * g, axis=1, keepdims=True)), MIN_NORM)
    maxn = (1.0 - EPS) / sc
    hy_ref[...] = jnp.where(gn > maxn, g / gn * maxn, g)


def _node_tables(curv11, hidden, Ws_w, br):
    n, d = hidden.shape
    return pl.pallas_call(
        _node_tables_body,
        grid=(n // br,),
        in_specs=[
            pl.BlockSpec((1, 1), lambda i: (0, 0)),
            pl.BlockSpec((br, d), lambda i: (i, 0)),
            pl.BlockSpec(Ws_w.shape, lambda i: (0, 0)),
        ],
        out_specs=[
            pl.BlockSpec((br, Ws_w.shape[0]), lambda i: (i, 0)),
            pl.BlockSpec((br, d), lambda i: (i, 0)),
        ],
        out_shape=[
            jax.ShapeDtypeStruct((n, Ws_w.shape[0]), jnp.float32),
            jax.ShapeDtypeStruct((n, d), jnp.float32),
        ],
    )(curv11, hidden, Ws_w)


def _rela_tables(curv11, rela, Wr_w, Wqr_w, br):
    n, d = rela.shape
    a = Wr_w.shape[0]
    return pl.pallas_call(
        _rela_tables_body,
        grid=(n // br,),
        in_specs=[
            pl.BlockSpec((1, 1), lambda i: (0, 0)),
            pl.BlockSpec((br, d), lambda i: (i, 0)),
            pl.BlockSpec(Wr_w.shape, lambda i: (0, 0)),
            pl.BlockSpec(Wqr_w.shape, lambda i: (0, 0)),
        ],
        out_specs=[
            pl.BlockSpec((br, a), lambda i: (i, 0)),
            pl.BlockSpec((br, a), lambda i: (i, 0)),
            pl.BlockSpec((br, d), lambda i: (i, 0)),
        ],
        out_shape=[
            jax.ShapeDtypeStruct((n, a), jnp.float32),
            jax.ShapeDtypeStruct((n, a), jnp.float32),
            jax.ShapeDtypeStruct((n, d), jnp.float32),
        ],
    )(curv11, rela, Wr_w, Wqr_w)


def _make_gather_kernel(E, N, Vp, D, DR):
    ew = E // NW
    n_full = ew // CG
    tail = ew - n_full * CG
    mesh = plsc.VectorSubcoreMesh(core_axis_name="c", subcore_axis_name="s",
                                  num_cores=NC, num_subcores=NS)

    @functools.partial(
        pl.kernel,
        out_type=(
            jax.ShapeDtypeStruct((E, D), jnp.float32),
            jax.ShapeDtypeStruct((E, D), jnp.float32),
            jax.ShapeDtypeStruct((E, D), jnp.float32),
            jax.ShapeDtypeStruct((E, D), jnp.float32),
        ),
        mesh=mesh,
        scratch_types=[
            pltpu.VMEM((CG,), jnp.int32),
            pltpu.VMEM((CG,), jnp.int32),
            pltpu.VMEM((CG,), jnp.int32),
            pltpu.VMEM((CG,), jnp.int32),
            pltpu.VMEM((CG, D), jnp.float32),
            pltpu.VMEM((CG, D), jnp.float32),
            pltpu.VMEM((CG, D), jnp.float32),
            pltpu.VMEM((CG, D), jnp.float32),
            pltpu.VMEM((CG, D), jnp.float32),
            pltpu.VMEM((CG, D), jnp.float32),
            pltpu.SemaphoreType.DMA,
        ],
        compiler_params=pltpu.CompilerParams(use_tc_tiling_on_sc=False),
    )
    def gather_k(sub_h, rel_h, ridx_h, qrel_h, p_h, q_h, r_h, hx_h, hy_h, qrp_h,
                 bg_h, xg_h, yg_h, qrpg_h,
                 subv, relv, ridxv, qqv, bp, bq, br_, bx, by, bqrp, sem):
        wid = lax.axis_index("s") * NC + lax.axis_index("c")
        base = wid * ew

        def do_chunk(off, cg):
            sl = pl.ds(0, cg)
            pltpu.sync_copy(sub_h.at[pl.ds(off, cg)], subv.at[sl])
            pltpu.sync_copy(rel_h.at[pl.ds(off, cg)], relv.at[sl])
            pltpu.sync_copy(ridx_h.at[pl.ds(off, cg)], ridxv.at[sl])
            pltpu.async_copy(qrel_h.at[ridxv.at[sl]], qqv.at[sl], sem).wait()
            cps = [
                pltpu.async_copy(p_h.at[subv.at[sl]], bp.at[sl], sem),
                pltpu.async_copy(q_h.at[relv.at[sl]], bq.at[sl], sem),
                pltpu.async_copy(r_h.at[qqv.at[sl]], br_.at[sl], sem),
                pltpu.async_copy(hx_h.at[subv.at[sl]], bx.at[sl], sem),
                pltpu.async_copy(hy_h.at[relv.at[sl]], by.at[sl], sem),
                pltpu.async_copy(qrp_h.at[ridxv.at[sl]], bqrp.at[sl], sem),
            ]
            for cp in cps:
                cp.wait()

            def addrow(r, _):
                for k in range(D // 16):
                    ls = pl.ds(k * 16, 16)
                    bp[r, ls] = bp[r, ls] + bq[r, ls] + br_[r, ls]
                return 0

            lax.fori_loop(0, cg, addrow, 0)

            ods = pl.ds(off, cg)
            wps = [
                pltpu.async_copy(bp.at[sl], bg_h.at[ods], sem),
                pltpu.async_copy(bx.at[sl], xg_h.at[ods], sem),
                pltpu.async_copy(by.at[sl], yg_h.at[ods], sem),
                pltpu.async_copy(bqrp.at[sl], qrpg_h.at[ods], sem),
            ]
            for cp in wps:
                cp.wait()

        def body(i, _):
            do_chunk(base + i * CG, CG)
            return 0

        lax.fori_loop(0, n_full, body, 0)
        if tail:
            do_chunk(base + n_full * CG, tail)

    return gather_k


def _edge_body(scal_ref, attnw_ref, attnb_ref, wqrb_ref, walw_ref, msgw_ref,
               bg_ref, xg_ref, yg_ref, qrp_ref, er_ref, out_ref, wout_ref):
    c = jnp.maximum(scal_ref[0, 0], MINC)
    walpha_b = scal_ref[0, 1]
    msg_b = scal_ref[0, 2]
    sc = jnp.sqrt(c)

    er = er_ref[...]
    be_, dr_ = er.shape
    rc = jnp.clip(er * qrp_ref[:, :dr_], -1.0, 1.0)
    t1 = lax.dot_general(rc, attnw_ref[...], (((1,), (1,)), ((), ())),
                         preferred_element_type=jnp.float32)
    scale = 2.0 * jax.nn.sigmoid(t1 + attnb_ref[...])
    base = bg_ref[...] + wqrb_ref[...]
    feat = scale * base
    logit = jnp.clip(
        jnp.sum(jax.nn.relu(feat) * walw_ref[...], axis=1, keepdims=True) + walpha_b,
        -MAXL, MAXL)
    w = jnp.exp(logit)
    gate = 2.0 * jax.nn.sigmoid(
        jnp.sum(rc * msgw_ref[...], axis=1, keepdims=True) + msg_b)

    x = xg_ref[...]
    y = yg_ref[...]
    x2 = jnp.sum(x * x, axis=1, keepdims=True)
    y2 = jnp.sum(y * y, axis=1, keepdims=True)
    xy = jnp.sum(x * y, axis=1, keepdims=True)
    cxy2 = 2.0 * c * xy
    a = 1.0 + cxy2 + c * y2
    b = 1.0 - c * x2
    denc = jnp.maximum(1.0 + cxy2 + (c * c) * (x2 * y2), MIN_NORM)
    rden = 1.0 / denc
    nm2 = (a * a) * x2 + (2.0 * a) * (b * xy) + (b * b) * y2
    nm = jnp.sqrt(jnp.maximum(nm2, 0.0)) * rden
    nmc = jnp.maximum(nm, MIN_NORM)
    maxn = (1.0 - EPS) / sc
    p = jnp.where(nmc > maxn, maxn / nmc, 1.0)
    yn = jnp.maximum(p * nm, MIN_NORM)
    z = jnp.minimum(sc * yn, 1.0 - EPS)
    t = 0.5 * (jnp.log1p(z) - jnp.log1p(-z)) / (sc * yn)
    wg = w * gate * t * p * rden
    wA = wg * a
    wB = wg * b
    out_ref[...] = wA * x + wB * y
    wout_ref[...] = jnp.reshape(w, (1, 1, be_))


def _edge_stage(scal, attn_w, attn_b, wqr_b, wal_w, msg_w,
                bg, xg, yg, qrpg, er, be):
    E, D = bg.shape
    DR = er.shape[1]
    full = lambda a: pl.BlockSpec(a.shape, lambda i: tuple(0 for _ in a.shape))
    blk = lambda d_: pl.BlockSpec((be, d_), lambda i: (i, 0))
    return pl.pallas_call(
        _edge_body,
        grid=(E // be,),
        in_specs=[full(scal), full(attn_w), full(attn_b), full(wqr_b),
                  full(wal_w), full(msg_w),
                  blk(D), blk(D), blk(D), blk(D), blk(DR)],
        out_specs=[pl.BlockSpec((be, MW), lambda i: (i, 0)),
                   pl.BlockSpec((1, 1, be), lambda i: (i, 0, 0))],
        out_shape=[jax.ShapeDtypeStruct((E, MW), jnp.float32),
                   jax.ShapeDtypeStruct((E // be, 1, be), jnp.float32)],
    )(scal, attn_w, attn_b, wqr_b, wal_w, msg_w, bg, xg, yg, qrpg, er)


def _make_scatter_kernel(E, N):
    HN = N // NC
    ew = E // NS
    CB = 400
    SUB = 80
    n_sub = CB // SUB
    n_chunk = ew // CB
    assert n_chunk * CB == ew
    rows_lo = (HN // NS) // 8 * 8
    rows_hi = HN - rows_lo * (NS - 1) + 8
    mesh = plsc.VectorSubcoreMesh(core_axis_name="c", subcore_axis_name="s",
                                  num_cores=NC, num_subcores=NS)

    @functools.partial(
        pl.kernel,
        out_type=(jax.ShapeDtypeStruct((N, MW), jnp.float32),
                  jax.ShapeDtypeStruct((N,), jnp.float32)),
        mesh=mesh,
        scratch_types=[
            pltpu.VMEM((CB, MW), jnp.float32),
            pltpu.VMEM((CB,), jnp.float32),
            pltpu.VMEM((n_sub, SUB), jnp.int32),
            pltpu.VMEM_SHARED((HN + 8, MW), jnp.float32),
            pltpu.VMEM_SHARED((HN + 8,), jnp.float32),
            pltpu.SemaphoreType.DMA,
        ],
        compiler_params=pltpu.CompilerParams(use_tc_tiling_on_sc=False),
    )
    def scatter_k(m_h, w_h, obj2_h, zu_h, za_h, up_h, ap_h, mb, wv, objv,
                  ush, ash, sem):
        cid = lax.axis_index("c")
        sid = lax.axis_index("s")
        lo = cid * HN
        base = sid * ew

        @pl.when(sid < NS - 1)
        def _():
            rsl = pl.ds(sid * rows_lo, rows_lo)
            pltpu.sync_copy(zu_h.at[rsl], ush.at[rsl])
            pltpu.sync_copy(za_h.at[rsl], ash.at[rsl])

        @pl.when(sid == NS - 1)
        def _():
            rsl = pl.ds((NS - 1) * rows_lo, rows_hi)
            pltpu.sync_copy(zu_h.at[rsl], ush.at[rsl])
            pltpu.sync_copy(za_h.at[rsl], ash.at[rsl])

        plsc.subcore_barrier()

        def body(i, _):
            off = base + i * CB
            pltpu.sync_copy(m_h.at[pl.ds(off, CB)], mb)
            pltpu.sync_copy(w_h.at[pl.ds(off, CB)], wv)
            pltpu.sync_copy(obj2_h.at[pl.ds(off // SUB, n_sub)], objv)
            trash = HN + (sid & 7)
            for j in range(n_sub):
                for k in range(SUB // 16):
                    o = objv[j, pl.ds(k * 16, 16)] - lo
                    ok = (o >= 0) & (o < HN)
                    objv[j, pl.ds(k * 16, 16)] = jnp.where(ok, o, trash)
                pltpu.sync_copy(mb.at[pl.ds(j * SUB, SUB)],
                                ush.at[objv.at[j]], add=True)
                pltpu.sync_copy(wv.at[pl.ds(j * SUB, SUB)],
                                ash.at[objv.at[j]], add=True)
            return 0

        lax.fori_loop(0, n_chunk, body, 0)
        plsc.subcore_barrier()

        @pl.when(sid < NS - 1)
        def _():
            rsl = pl.ds(sid * rows_lo, rows_lo)
            osl = pl.ds(lo + sid * rows_lo, rows_lo)
            pltpu.sync_copy(ush.at[rsl], up_h.at[osl])
            pltpu.sync_copy(ash.at[rsl], ap_h.at[osl])

        @pl.when(sid == NS - 1)
        def _():
            nlast = HN - (NS - 1) * rows_lo
            rsl = pl.ds((NS - 1) * rows_lo, nlast)
            osl = pl.ds(lo + (NS - 1) * rows_lo, nlast)
            pltpu.sync_copy(ush.at[rsl], up_h.at[osl])
            pltpu.sync_copy(ash.at[rsl], ap_h.at[osl])

    return scatter_k


def _final_body(u0_ref, u1_ref, a0_ref, a1_ref, whp_ref, out_ref):
    u = u0_ref[...] + u1_ref[...]
    a = a0_ref[...] + a1_ref[...]
    asum = jnp.maximum(jnp.reshape(a, (u.shape[0], 1)), MIN_NORM)
    o = lax.dot_general(u, whp_ref[...], (((1,), (0,)), ((), ())),
                        preferred_element_type=jnp.float32)
    out_ref[...] = o / asum


def _final_stage(u0, u1, a0, a1, whp, br):
    N, _ = u0.shape
    D = whp.shape[1]
    a03 = a0.reshape(N // br, 1, br)
    a13 = a1.reshape(N // br, 1, br)
    return pl.pallas_call(
        _final_body,
        grid=(N // br,),
        in_specs=[
            pl.BlockSpec((br, MW), lambda i: (i, 0)),
            pl.BlockSpec((br, MW), lambda i: (i, 0)),
            pl.BlockSpec((1, 1, br), lambda i: (i, 0, 0)),
            pl.BlockSpec((1, 1, br), lambda i: (i, 0, 0)),
            pl.BlockSpec(whp.shape, lambda i: (0, 0)),
        ],
        out_specs=pl.BlockSpec((br, D), lambda i: (i, 0)),
        out_shape=jax.ShapeDtypeStruct((N, D), jnp.float32),
    )(u0, u1, a03, a13, whp)


def kernel(q_sub, q_rel, hidden, edges, nodes, old_nodes_new_idx, batchsize,
           curvature, edge_rule, query_rule_pref, rela_embed, Ws_w, Wr_w,
           Wqr_w, Wqr_b, walpha_w, walpha_b, Wh_w, rule_attn_w, rule_attn_b,
           rule_msg_w, rule_msg_b):
    E = edges.shape[0]
    N, D = hidden.shape
    V = rela_embed.shape[0]
    DR = edge_rule.shape[1]
    A = Ws_w.shape[0]
    Vp = 10240 if V <= 10240 else ((V + 1023) // 1024) * 1024

    sub = edges[:, 4]
    rel = edges[:, 2]
    obj = edges[:, 5]
    ridx = edges[:, 0]
    curv11 = curvature.reshape(1, 1)
    rela_pad = jnp.zeros((Vp, D), jnp.float32).at[:V].set(rela_embed)

    P, HX = _node_tables(curv11, hidden, Ws_w, br=1000)
    Q, R, HY = _rela_tables(curv11, rela_pad, Wr_w, Wqr_w, br=1024)

    qrp_pad = jnp.zeros((N, D), jnp.float32).at[:, :DR].set(query_rule_pref)
    scal = jnp.concatenate([curvature, walpha_b, rule_msg_b,
                            jnp.zeros((1,), jnp.float32)]).reshape(1, 4)
    E2 = E // 2
    gather_k = _make_gather_kernel(E2, N, Vp, D, DR)
    scatter_k = _make_scatter_kernel(E2, N)
    obj2 = obj.reshape(E // 80, 80)
    zu = jnp.zeros((N // NC + 8, MW), jnp.float32)
    za = jnp.zeros((N // NC + 8,), jnp.float32)

    parts = []
    for h in range(2):
        sl = slice(h * E2, (h + 1) * E2)
        bg, xg, yg, qrpg = gather_k(sub[sl], rel[sl], ridx[sl], q_rel,
                                    P, Q, R, HX, HY, qrp_pad)
        m_rows, w3 = _edge_stage(scal, rule_attn_w, rule_attn_b.reshape(1, A),
                                 Wqr_b.reshape(1, A), walpha_w, rule_msg_w,
                                 bg, xg, yg, qrpg, edge_rule[sl], be=3200)
        w_e = w3.reshape(E2)
        o2 = obj2[h * (E2 // 80):(h + 1) * (E2 // 80)]
        parts.append(scatter_k(m_rows, w_e, o2, zu, za))

    (u0, a0), (u1, a1) = parts
    return _final_stage(u0, u1, a0, a1, Wh_w.T, br=1000)


# speedup vs baseline: 7.1433x; 1.0620x over previous
import functools

import jax
import jax.numpy as jnp
from jax import lax
from jax.experimental import pallas as pl
from jax.experimental.pallas import tpu as pltpu
from jax.experimental.pallas import tpu_sc as plsc

MIN_NORM = 1e-15
MAXL = 50.0
EPS = 0.004
MINC = 1e-6

NC, NS = 2, 16
NW = NC * NS
CG = 128
MW = 128


def _node_tables_body(curv_ref, h_ref, w_ref, p_ref, hx_ref):
    h = h_ref[...]
    p_ref[...] = lax.dot_general(h, w_ref[...], (((1,), (1,)), ((), ())),
                                 preferred_element_type=jnp.float32)
    c = jnp.maximum(curv_ref[0, 0], MINC)
    sc = jnp.sqrt(c)
    un = jnp.maximum(jnp.sqrt(jnp.sum(h * h, axis=1, keepdims=True)), MIN_NORM)
    g = jnp.tanh(jnp.clip(sc * un, -15.0, 15.0)) * h / (sc * un)
    gn = jnp.maximum(jnp.sqrt(jnp.sum(g * g, axis=1, keepdims=True)), MIN_NORM)
    maxn = (1.0 - EPS) / sc
    hx_ref[...] = jnp.where(gn > maxn, g / gn * maxn, g)


def _rela_tables_body(curv_ref, h_ref, wr_ref, wqr_ref, q_ref, r_ref, hy_ref):
    h = h_ref[...]
    q_ref[...] = lax.dot_general(h, wr_ref[...], (((1,), (1,)), ((), ())),
                                 preferred_element_type=jnp.float32)
    r_ref[...] = lax.dot_general(h, wqr_ref[...], (((1,), (1,)), ((), ())),
                                 preferred_element_type=jnp.float32)
    c = jnp.maximum(curv_ref[0, 0], MINC)
    sc = jnp.sqrt(c)
    un = jnp.maximum(jnp.sqrt(jnp.sum(h * h, axis=1, keepdims=True)), MIN_NORM)
    g = jnp.tanh(jnp.clip(sc * un, -15.0, 15.0)) * h / (sc * un)
    gn = jnp.maximum(jnp.sqrt(jnp.sum(g * g, axis=1, keepdims=True)), MIN_NORM)
    maxn = (1.0 - EPS) / sc
    hy_ref[...] = jnp.where(gn > maxn, g / gn * maxn, g)


def _node_tables(curv11, hidden, Ws_w, br):
    n, d = hidden.shape
    return pl.pallas_call(
        _node_tables_body,
        grid=(n // br,),
        in_specs=[
            pl.BlockSpec((1, 1), lambda i: (0, 0)),
            pl.BlockSpec((br, d), lambda i: (i, 0)),
            pl.BlockSpec(Ws_w.shape, lambda i: (0, 0)),
        ],
        out_specs=[
            pl.BlockSpec((br, Ws_w.shape[0]), lambda i: (i, 0)),
            pl.BlockSpec((br, d), lambda i: (i, 0)),
        ],
        out_shape=[
            jax.ShapeDtypeStruct((n, Ws_w.shape[0]), jnp.float32),
            jax.ShapeDtypeStruct((n, d), jnp.float32),
        ],
    )(curv11, hidden, Ws_w)


def _rela_tables(curv11, rela, Wr_w, Wqr_w, br):
    n, d = rela.shape
    a = Wr_w.shape[0]
    return pl.pallas_call(
        _rela_tables_body,
        grid=(n // br,),
        in_specs=[
            pl.BlockSpec((1, 1), lambda i: (0, 0)),
            pl.BlockSpec((br, d), lambda i: (i, 0)),
            pl.BlockSpec(Wr_w.shape, lambda i: (0, 0)),
            pl.BlockSpec(Wqr_w.shape, lambda i: (0, 0)),
        ],
        out_specs=[
            pl.BlockSpec((br, a), lambda i: (i, 0)),
            pl.BlockSpec((br, a), lambda i: (i, 0)),
            pl.BlockSpec((br, d), lambda i: (i, 0)),
        ],
        out_shape=[
            jax.ShapeDtypeStruct((n, a), jnp.float32),
            jax.ShapeDtypeStruct((n, a), jnp.float32),
            jax.ShapeDtypeStruct((n, d), jnp.float32),
        ],
    )(curv11, rela, Wr_w, Wqr_w)


def _make_gather_kernel(E, N, Vp, D, DR):
    CGd = 64
    ew = E // NW
    n_full = ew // CGd
    tail = ew - n_full * CGd
    assert n_full % 2 == 0
    npairs = n_full // 2
    mesh = plsc.VectorSubcoreMesh(core_axis_name="c", subcore_axis_name="s",
                                  num_cores=NC, num_subcores=NS)

    @functools.partial(
        pl.kernel,
        out_type=(
            jax.ShapeDtypeStruct((E, D), jnp.float32),
            jax.ShapeDtypeStruct((E, D), jnp.float32),
            jax.ShapeDtypeStruct((E, D), jnp.float32),
            jax.ShapeDtypeStruct((E, D), jnp.float32),
        ),
        mesh=mesh,
        scratch_types=[
            pltpu.VMEM((2, CGd), jnp.int32),
            pltpu.VMEM((2, CGd), jnp.int32),
            pltpu.VMEM((2, CGd), jnp.int32),
            pltpu.VMEM((2, CGd), jnp.int32),
            pltpu.VMEM((2, CGd, D), jnp.float32),
            pltpu.VMEM((2, CGd, D), jnp.float32),
            pltpu.VMEM((2, CGd, D), jnp.float32),
            pltpu.VMEM((2, CGd, D), jnp.float32),
            pltpu.VMEM((2, CGd, D), jnp.float32),
            pltpu.VMEM((2, CGd, D), jnp.float32),
            pltpu.SemaphoreType.DMA,
            pltpu.SemaphoreType.DMA,
            pltpu.SemaphoreType.DMA,
            pltpu.SemaphoreType.DMA,
        ],
        compiler_params=pltpu.CompilerParams(use_tc_tiling_on_sc=False),
    )
    def gather_k(sub_h, rel_h, ridx_h, qrel_h, p_h, q_h, r_h, hx_h, hy_h, qrp_h,
                 bg_h, xg_h, yg_h, qrpg_h,
                 subv, relv, ridxv, qqv, bp, bq, br_, bx, by, bqrp,
                 gs0, gs1, ws0, ws1):
        wid = lax.axis_index("s") * NC + lax.axis_index("c")
        base = wid * ew
        gsem = (gs0, gs1)
        wsem = (ws0, ws1)

        def idx_phase(off, b, cg):
            sl = pl.ds(0, cg)
            pltpu.sync_copy(sub_h.at[pl.ds(off, cg)], subv.at[b, sl])
            pltpu.sync_copy(rel_h.at[pl.ds(off, cg)], relv.at[b, sl])
            pltpu.sync_copy(ridx_h.at[pl.ds(off, cg)], ridxv.at[b, sl])
            pltpu.async_copy(qrel_h.at[ridxv.at[b, sl]], qqv.at[b, sl],
                             gsem[b]).wait()

        def issue_gathers(b):
            sl = pl.ds(0, CGd)
            s = gsem[b]
            pltpu.async_copy(p_h.at[subv.at[b, sl]], bp.at[b], s)
            pltpu.async_copy(q_h.at[relv.at[b, sl]], bq.at[b], s)
            pltpu.async_copy(r_h.at[qqv.at[b, sl]], br_.at[b], s)
            pltpu.async_copy(hx_h.at[subv.at[b, sl]], bx.at[b], s)
            pltpu.async_copy(hy_h.at[relv.at[b, sl]], by.at[b], s)
            pltpu.async_copy(qrp_h.at[ridxv.at[b, sl]], bqrp.at[b], s)

        def wait_gathers(b):
            hsl = pl.ds(0, CGd)
            s = gsem[b]
            pltpu.make_async_copy(p_h.at[hsl], bp.at[b], s).wait()
            pltpu.make_async_copy(q_h.at[hsl], bq.at[b], s).wait()
            pltpu.make_async_copy(r_h.at[hsl], br_.at[b], s).wait()
            pltpu.make_async_copy(hx_h.at[hsl], bx.at[b], s).wait()
            pltpu.make_async_copy(hy_h.at[hsl], by.at[b], s).wait()
            pltpu.make_async_copy(qrp_h.at[hsl], bqrp.at[b], s).wait()

        def do_adds(b, cg):
            def addrow(r, _):
                for k in range(D // 16):
                    ls = pl.ds(k * 16, 16)
                    bp[b, r, ls] = bp[b, r, ls] + bq[b, r, ls] + br_[b, r, ls]
                return 0

            lax.fori_loop(0, cg, addrow, 0)

        def issue_writes(off, b):
            ods = pl.ds(off, CGd)
            s = wsem[b]
            pltpu.async_copy(bp.at[b], bg_h.at[ods], s)
            pltpu.async_copy(bx.at[b], xg_h.at[ods], s)
            pltpu.async_copy(by.at[b], yg_h.at[ods], s)
            pltpu.async_copy(bqrp.at[b], qrpg_h.at[ods], s)

        def wait_writes(b):
            hsl = pl.ds(0, CGd)
            s = wsem[b]
            pltpu.make_async_copy(bp.at[b], bg_h.at[hsl], s).wait()
            pltpu.make_async_copy(bx.at[b], xg_h.at[hsl], s).wait()
            pltpu.make_async_copy(by.at[b], yg_h.at[hsl], s).wait()
            pltpu.make_async_copy(bqrp.at[b], qrpg_h.at[hsl], s).wait()

        def pair(i, _):
            off0 = base + (2 * i) * CGd
            off1 = off0 + CGd

            @pl.when(i > 0)
            def _():
                wait_writes(0)

            idx_phase(off0, 0, CGd)
            issue_gathers(0)

            @pl.when(i > 0)
            def _():
                wait_gathers(1)
                do_adds(1, CGd)
                issue_writes(off0 - CGd, 1)
                wait_writes(1)

            idx_phase(off1, 1, CGd)
            issue_gathers(1)
            wait_gathers(0)
            do_adds(0, CGd)
            issue_writes(off0, 0)
            return 0

        lax.fori_loop(0, npairs, pair, 0)
        wait_gathers(1)
        do_adds(1, CGd)
        issue_writes(base + (n_full - 1) * CGd, 1)
        wait_writes(0)
        wait_writes(1)

        if tail:
            off = base + n_full * CGd
            sl = pl.ds(0, tail)
            idx_phase(off, 0, tail)
            s = gsem[0]
            cps = [
                pltpu.async_copy(p_h.at[subv.at[0, sl]], bp.at[0, sl], s),
                pltpu.async_copy(q_h.at[relv.at[0, sl]], bq.at[0, sl], s),
                pltpu.async_copy(r_h.at[qqv.at[0, sl]], br_.at[0, sl], s),
                pltpu.async_copy(hx_h.at[subv.at[0, sl]], bx.at[0, sl], s),
                pltpu.async_copy(hy_h.at[relv.at[0, sl]], by.at[0, sl], s),
                pltpu.async_copy(qrp_h.at[ridxv.at[0, sl]], bqrp.at[0, sl], s),
            ]
            for cp in cps:
                cp.wait()
            do_adds(0, tail)
            ods = pl.ds(off, tail)
            wps = [
                pltpu.async_copy(bp.at[0, sl], bg_h.at[ods], s),
                pltpu.async_copy(bx.at[0, sl], xg_h.at[ods], s),
                pltpu.async_copy(by.at[0, sl], yg_h.at[ods], s),
                pltpu.async_copy(bqrp.at[0, sl], qrpg_h.at[ods], s),
            ]
            for cp in wps:
                cp.wait()

    return gather_k


def _edge_body(scal_ref, attnw_ref, attnb_ref, wqrb_ref, walw_ref, msgw_ref,
               bg_ref, xg_ref, yg_ref, qrp_ref, er_ref, out_ref, wout_ref):
    c = jnp.maximum(scal_ref[0, 0], MINC)
    walpha_b = scal_ref[0, 1]
    msg_b = scal_ref[0, 2]
    sc = jnp.sqrt(c)

    er = er_ref[...]
    be_, dr_ = er.shape
    rc = jnp.clip(er * qrp_ref[:, :dr_], -1.0, 1.0)
    t1 = lax.dot_general(rc, attnw_ref[...], (((1,), (1,)), ((), ())),
                         preferred_element_type=jnp.float32)
    scale = 2.0 * jax.nn.sigmoid(t1 + attnb_ref[...])
    base = bg_ref[...] + wqrb_ref[...]
    feat = scale * base
    logit = jnp.clip(
        jnp.sum(jax.nn.relu(feat) * walw_ref[...], axis=1, keepdims=True) + walpha_b,
        -MAXL, MAXL)
    w = jnp.exp(logit)
    gate = 2.0 * jax.nn.sigmoid(
        jnp.sum(rc * msgw_ref[...], axis=1, keepdims=True) + msg_b)

    x = xg_ref[...].astype(jnp.float32)
    y = yg_ref[...].astype(jnp.float32)
    x2 = jnp.sum(x * x, axis=1, keepdims=True)
    y2 = jnp.sum(y * y, axis=1, keepdims=True)
    xy = jnp.sum(x * y, axis=1, keepdims=True)
    cxy2 = 2.0 * c * xy
    a = 1.0 + cxy2 + c * y2
    b = 1.0 - c * x2
    denc = jnp.maximum(1.0 + cxy2 + (c * c) * (x2 * y2), MIN_NORM)
    rden = 1.0 / denc
    nm2 = (a * a) * x2 + (2.0 * a) * (b * xy) + (b * b) * y2
    nm = jnp.sqrt(jnp.maximum(nm2, 0.0)) * rden
    nmc = jnp.maximum(nm, MIN_NORM)
    maxn = (1.0 - EPS) / sc
    p = jnp.where(nmc > maxn, maxn / nmc, 1.0)
    yn = jnp.maximum(p * nm, MIN_NORM)
    z = jnp.minimum(sc * yn, 1.0 - EPS)
    t = 0.5 * (jnp.log1p(z) - jnp.log1p(-z)) / (sc * yn)
    wg = w * gate * t * p * rden
    wA = wg * a
    wB = wg * b
    out_ref[...] = wA * x + wB * y
    wout_ref[...] = jnp.reshape(w, (1, 1, be_))


def _edge_stage(scal, attn_w, attn_b, wqr_b, wal_w, msg_w,
                bg, xg, yg, qrpg, er, be):
    E, D = bg.shape
    DR = er.shape[1]
    full = lambda a: pl.BlockSpec(a.shape, lambda i: tuple(0 for _ in a.shape))
    blk = lambda d_: pl.BlockSpec((be, d_), lambda i: (i, 0))
    return pl.pallas_call(
        _edge_body,
        grid=(E // be,),
        in_specs=[full(scal), full(attn_w), full(attn_b), full(wqr_b),
                  full(wal_w), full(msg_w),
                  blk(D), blk(D), blk(D), blk(D), blk(DR)],
        out_specs=[pl.BlockSpec((be, MW), lambda i: (i, 0)),
                   pl.BlockSpec((1, 1, be), lambda i: (i, 0, 0))],
        out_shape=[jax.ShapeDtypeStruct((E, MW), jnp.float32),
                   jax.ShapeDtypeStruct((E // be, 1, be), jnp.float32)],
    )(scal, attn_w, attn_b, wqr_b, wal_w, msg_w, bg, xg, yg, qrpg, er)


def _make_scatter_kernel(E, N):
    HN = N // NC
    ew = E // NS
    CB = 80
    SUB = 80
    n_sub = CB // SUB
    n_chunk = ew // CB
    assert n_chunk * CB == ew
    rows_lo = (HN // NS) // 8 * 8
    rows_hi = HN - rows_lo * (NS - 1) + 8
    mesh = plsc.VectorSubcoreMesh(core_axis_name="c", subcore_axis_name="s",
                                  num_cores=NC, num_subcores=NS)

    @functools.partial(
        pl.kernel,
        out_type=(jax.ShapeDtypeStruct((N, MW), jnp.float32),
                  jax.ShapeDtypeStruct((N,), jnp.float32)),
        mesh=mesh,
        scratch_types=[
            pltpu.VMEM((2, CB, MW), jnp.float32),
            pltpu.VMEM((2, CB), jnp.float32),
            pltpu.VMEM((2, n_sub, SUB), jnp.int32),
            pltpu.VMEM_SHARED((HN + 8, MW), jnp.float32),
            pltpu.VMEM_SHARED((HN + 8,), jnp.float32),
            pltpu.SemaphoreType.DMA,
            pltpu.SemaphoreType.DMA,
        ],
        compiler_params=pltpu.CompilerParams(use_tc_tiling_on_sc=False),
    )
    def scatter_k(m_h, w_h, obj2_h, zu_h, za_h, up_h, ap_h, mb, wv, objv,
                  ush, ash, ls0, ls1):
        cid = lax.axis_index("c")
        sid = lax.axis_index("s")
        lo = cid * HN
        base = sid * ew
        lsem = (ls0, ls1)

        def issue_loads(i, b):
            off = base + i * CB
            s = lsem[b]
            pltpu.async_copy(m_h.at[pl.ds(off, CB)], mb.at[b], s)
            pltpu.async_copy(w_h.at[pl.ds(off, CB)], wv.at[b], s)
            pltpu.async_copy(obj2_h.at[pl.ds(off // SUB, n_sub)], objv.at[b], s)

        def wait_loads(b):
            s = lsem[b]
            pltpu.make_async_copy(m_h.at[pl.ds(0, CB)], mb.at[b], s).wait()
            pltpu.make_async_copy(w_h.at[pl.ds(0, CB)], wv.at[b], s).wait()
            pltpu.make_async_copy(obj2_h.at[pl.ds(0, n_sub)], objv.at[b], s).wait()

        @pl.when(sid < NS - 1)
        def _():
            rsl = pl.ds(sid * rows_lo, rows_lo)
            pltpu.sync_copy(zu_h.at[rsl], ush.at[rsl])
            pltpu.sync_copy(za_h.at[rsl], ash.at[rsl])

        @pl.when(sid == NS - 1)
        def _():
            rsl = pl.ds((NS - 1) * rows_lo, rows_hi)
            pltpu.sync_copy(zu_h.at[rsl], ush.at[rsl])
            pltpu.sync_copy(za_h.at[rsl], ash.at[rsl])

        plsc.subcore_barrier()

        def process(i, b):
            wait_loads(b)

            @pl.when(i + 1 < n_chunk)
            def _():
                issue_loads(i + 1, 1 - b)

            trash = HN + (sid & 7)
            for j in range(n_sub):
                for k in range(SUB // 16):
                    o = objv[b, j, pl.ds(k * 16, 16)] - lo
                    ok = (o >= 0) & (o < HN)
                    objv[b, j, pl.ds(k * 16, 16)] = jnp.where(ok, o, trash)
                pltpu.sync_copy(mb.at[b, pl.ds(j * SUB, SUB)],
                                ush.at[objv.at[b, j]], add=True)
                pltpu.sync_copy(wv.at[b, pl.ds(j * SUB, SUB)],
                                ash.at[objv.at[b, j]], add=True)

        issue_loads(0, 0)

        def body(i, _):
            @pl.when(lax.rem(i, 2) == 0)
            def _():
                process(i, 0)

            @pl.when(lax.rem(i, 2) == 1)
            def _():
                process(i, 1)

            return 0

        lax.fori_loop(0, n_chunk, body, 0)
        plsc.subcore_barrier()

        @pl.when(sid < NS - 1)
        def _():
            rsl = pl.ds(sid * rows_lo, rows_lo)
            osl = pl.ds(lo + sid * rows_lo, rows_lo)
            pltpu.sync_copy(ush.at[rsl], up_h.at[osl])
            pltpu.sync_copy(ash.at[rsl], ap_h.at[osl])

        @pl.when(sid == NS - 1)
        def _():
            nlast = HN - (NS - 1) * rows_lo
            rsl = pl.ds((NS - 1) * rows_lo, nlast)
            osl = pl.ds(lo + (NS - 1) * rows_lo, nlast)
            pltpu.sync_copy(ush.at[rsl], up_h.at[osl])
            pltpu.sync_copy(ash.at[rsl], ap_h.at[osl])

    return scatter_k


def _final_body(u0_ref, u1_ref, a0_ref, a1_ref, whp_ref, out_ref):
    u = u0_ref[...] + u1_ref[...]
    a = a0_ref[...] + a1_ref[...]
    asum = jnp.maximum(jnp.reshape(a, (u.shape[0], 1)), MIN_NORM)
    o = lax.dot_general(u, whp_ref[...], (((1,), (0,)), ((), ())),
                        preferred_element_type=jnp.float32)
    out_ref[...] = o / asum


def _final_stage(u0, u1, a0, a1, whp, br):
    N, _ = u0.shape
    D = whp.shape[1]
    a03 = a0.reshape(N // br, 1, br)
    a13 = a1.reshape(N // br, 1, br)
    return pl.pallas_call(
        _final_body,
        grid=(N // br,),
        in_specs=[
            pl.BlockSpec((br, MW), lambda i: (i, 0)),
            pl.BlockSpec((br, MW), lambda i: (i, 0)),
            pl.BlockSpec((1, 1, br), lambda i: (i, 0, 0)),
            pl.BlockSpec((1, 1, br), lambda i: (i, 0, 0)),
            pl.BlockSpec(whp.shape, lambda i: (0, 0)),
        ],
        out_specs=pl.BlockSpec((br, D), lambda i: (i, 0)),
        out_shape=jax.ShapeDtypeStruct((N, D), jnp.float32),
    )(u0, u1, a03, a13, whp)


def kernel(q_sub, q_rel, hidden, edges, nodes, old_nodes_new_idx, batchsize,
           curvature, edge_rule, query_rule_pref, rela_embed, Ws_w, Wr_w,
           Wqr_w, Wqr_b, walpha_w, walpha_b, Wh_w, rule_attn_w, rule_attn_b,
           rule_msg_w, rule_msg_b):
    E = edges.shape[0]
    N, D = hidden.shape
    V = rela_embed.shape[0]
    DR = edge_rule.shape[1]
    A = Ws_w.shape[0]
    Vp = 10240 if V <= 10240 else ((V + 1023) // 1024) * 1024

    sub = edges[:, 4]
    rel = edges[:, 2]
    obj = edges[:, 5]
    ridx = edges[:, 0]
    curv11 = curvature.reshape(1, 1)
    rela_pad = jnp.zeros((Vp, D), jnp.float32).at[:V].set(rela_embed)

    P, HX = _node_tables(curv11, hidden, Ws_w, br=1000)
    Q, R, HY = _rela_tables(curv11, rela_pad, Wr_w, Wqr_w, br=1024)

    qrp_pad = jnp.zeros((N, D), jnp.float32).at[:, :DR].set(query_rule_pref)
    scal = jnp.concatenate([curvature, walpha_b, rule_msg_b,
                            jnp.zeros((1,), jnp.float32)]).reshape(1, 4)
    E2 = E // 2
    gather_k = _make_gather_kernel(E2, N, Vp, D, DR)
    scatter_k = _make_scatter_kernel(E2, N)
    obj2 = obj.reshape(E // 80, 80)
    zu = jnp.zeros((N // NC + 8, MW), jnp.float32)
    za = jnp.zeros((N // NC + 8,), jnp.float32)

    parts = []
    for h in range(2):
        sl = slice(h * E2, (h + 1) * E2)
        bg, xg, yg, qrpg = gather_k(sub[sl], rel[sl], ridx[sl], q_rel,
                                    P, Q, R, HX, HY, qrp_pad)
        m_rows, w3 = _edge_stage(scal, rule_attn_w, rule_attn_b.reshape(1, A),
                                 Wqr_b.reshape(1, A), walpha_w, rule_msg_w,
                                 bg, xg, yg, qrpg, edge_rule[sl], be=3200)
        w_e = w3.reshape(E2)
        o2 = obj2[h * (E2 // 80):(h + 1) * (E2 // 80)]
        parts.append(scatter_k(m_rows, w_e, o2, zu, za))

    (u0, a0), (u1, a1) = parts
    return _final_stage(u0, u1, a0, a1, Wh_w.T, br=1000)
